# baseline (speedup 1.0000x reference)
import jax
import jax.numpy as jnp
from jax.experimental import pallas as pl
from jax.experimental.pallas import tpu as pltpu
from jax.experimental.pallas import tpu_sc as plsc

_TILE = 512
_CHUNK = 2048
_GATHER_WINDOW = 256


def _vq_tc_kernel(z_ref, x2_ref, e2_ref, e_ref, idx_ref, loss_ref):
    z = z_ref[...]
    e = e_ref[...]
    dot2 = jnp.dot((2.0 * z).astype(jnp.bfloat16), e.astype(jnp.bfloat16),
                   preferred_element_type=jnp.float32)
    x2 = x2_ref[...]
    e2 = e2_ref[...]
    d = (x2 - dot2) + e2
    K = d.shape[1]
    kiota = jax.lax.broadcasted_iota(jnp.int32, d.shape, 1)
    big = jnp.int32(K)
    bv = None
    for c in range(K // _CHUNK):
        dc = d[:, c * _CHUNK:(c + 1) * _CHUNK]
        cm = jnp.min(dc, axis=1, keepdims=True)
        ci = jnp.min(jnp.where(dc == cm, kiota[:, c * _CHUNK:(c + 1) * _CHUNK],
                               big), axis=1, keepdims=True)
        if bv is None:
            bv, bi, wv = cm, ci, cm
        else:
            steal = cm < bv
            bv = jnp.where(steal, cm, bv)
            bi = jnp.where(steal, ci, bi)
            wv = jnp.where(steal, cm, wv)
        bv = bv.astype(jnp.bfloat16).astype(jnp.float32)
    idx_ref[0, 0, :] = bi[:, 0]
    loss_ref[0, 0, 0] = jnp.sum(wv)


def _sc_gather(et, idx_row, n, c_dim):
    mesh = plsc.VectorSubcoreMesh(core_axis_name="core",
                                  subcore_axis_name="subcore")

    @pl.kernel(out_type=jax.ShapeDtypeStruct((n, c_dim), jnp.float32),
               mesh=mesh)
    def sc_kernel(et_hbm, i_hbm, o_hbm):
        def body(i_vmem, o_vmem):
            pltpu.sync_copy(et_hbm.at[i_vmem.at[0]], o_vmem)

        pltpu.emit_pipeline(
            body,
            grid=(n // _GATHER_WINDOW,),
            in_specs=[pl.BlockSpec((1, _GATHER_WINDOW),
                                   index_map=lambda i: (0, i))],
            out_specs=[pl.BlockSpec((_GATHER_WINDOW, c_dim),
                                    index_map=lambda i: (i, 0))],
            core_axis_name=("core", "subcore"),
            dimension_semantics=(pltpu.PARALLEL,),
        )(i_hbm, o_hbm)

    return sc_kernel(et, idx_row)


def kernel(x, embedding_table):
    B, C, H, W = x.shape
    K = embedding_table.shape[1]
    N = B * H * W
    flat_x = jnp.transpose(x, (0, 2, 3, 1)).reshape(N, C)
    x2 = (flat_x ** 2).sum(axis=1, keepdims=True)
    e2 = (embedding_table ** 2).sum(axis=0, keepdims=True)

    nblocks = N // _TILE
    idx3, loss_sum = pl.pallas_call(
        _vq_tc_kernel,
        grid=(nblocks,),
        in_specs=[
            pl.BlockSpec((_TILE, C), lambda i: (i, 0)),
            pl.BlockSpec((_TILE, 1), lambda i: (i, 0)),
            pl.BlockSpec((1, K), lambda i: (0, 0)),
            pl.BlockSpec((C, K), lambda i: (0, 0)),
        ],
        out_specs=[
            pl.BlockSpec((1, 1, _TILE), lambda i: (i, 0, 0)),
            pl.BlockSpec((1, 1, 1), lambda i: (i, 0, 0),
                         memory_space=pltpu.SMEM),
        ],
        out_shape=[
            jax.ShapeDtypeStruct((nblocks, 1, _TILE), jnp.int32),
            jax.ShapeDtypeStruct((nblocks, 1, 1), jnp.float32),
        ],
        compiler_params=pltpu.CompilerParams(
            dimension_semantics=("parallel",)),
    )(flat_x, x2, e2, embedding_table)

    et = jnp.pad(embedding_table.T, ((0, 0), (0, 128 - C)))
    q_flat = _sc_gather(et, idx3.reshape(1, N), N, 128)[:, :C]

    quantized = jnp.transpose(q_flat.reshape(B, H, W, C), (0, 3, 1, 2))
    loss = loss_sum.sum() / jnp.float32(N * C)
    indices = idx3.reshape(B, H * W)
    return (quantized, loss, loss, indices)


# speedup vs baseline: 1.1280x; 1.1280x over previous
import jax
import jax.numpy as jnp
from jax.experimental import pallas as pl
from jax.experimental.pallas import tpu as pltpu
from jax.experimental.pallas import tpu_sc as plsc

_TILE = 512
_CHUNK = 2048
_GATHER_WINDOW = 256


def _vq_tc_kernel(z_ref, x2_ref, e2_ref, e_ref, idx_ref, loss_ref):
    z = z_ref[...]
    e = e_ref[...]
    dot2 = jnp.dot((2.0 * z).astype(jnp.bfloat16), e.astype(jnp.bfloat16),
                   preferred_element_type=jnp.float32)
    x2 = x2_ref[...]
    e2 = e2_ref[...]
    d = (x2 - dot2) + e2
    K = d.shape[1]
    kiota = jax.lax.broadcasted_iota(jnp.int32, d.shape, 1).astype(jnp.float32)
    big = jnp.float32(K)
    bv = None
    for c in range(K // _CHUNK):
        dc = d[:, c * _CHUNK:(c + 1) * _CHUNK]
        cm = jnp.min(dc, axis=1, keepdims=True)
        ci = jnp.min(jnp.where(dc == cm, kiota[:, c * _CHUNK:(c + 1) * _CHUNK],
                               big), axis=1, keepdims=True)
        if bv is None:
            bv, bi, wv = cm, ci, cm
        else:
            steal = cm < bv
            bv = jnp.where(steal, cm, bv)
            bi = jnp.where(steal, ci, bi)
            wv = jnp.where(steal, cm, wv)
        bv = bv.astype(jnp.bfloat16).astype(jnp.float32)
    idx_ref[0, 0, :] = bi[:, 0].astype(jnp.int32)
    loss_ref[0, 0, 0] = jnp.sum(wv)


def _sc_gather(et, idx_row, n, c_dim):
    mesh = plsc.VectorSubcoreMesh(core_axis_name="core",
                                  subcore_axis_name="subcore")

    @pl.kernel(out_type=jax.ShapeDtypeStruct((n, c_dim), jnp.float32),
               mesh=mesh)
    def sc_kernel(et_hbm, i_hbm, o_hbm):
        def body(i_vmem, o_vmem):
            pltpu.sync_copy(et_hbm.at[i_vmem.at[0]], o_vmem)

        pltpu.emit_pipeline(
            body,
            grid=(n // _GATHER_WINDOW,),
            in_specs=[pl.BlockSpec((1, _GATHER_WINDOW),
                                   index_map=lambda i: (0, i))],
            out_specs=[pl.BlockSpec((_GATHER_WINDOW, c_dim),
                                    index_map=lambda i: (i, 0))],
            core_axis_name=("core", "subcore"),
            dimension_semantics=(pltpu.PARALLEL,),
        )(i_hbm, o_hbm)

    return sc_kernel(et, idx_row)


def kernel(x, embedding_table):
    B, C, H, W = x.shape
    K = embedding_table.shape[1]
    N = B * H * W
    flat_x = jnp.transpose(x, (0, 2, 3, 1)).reshape(N, C)
    x2 = (flat_x ** 2).sum(axis=1, keepdims=True)
    e2 = (embedding_table ** 2).sum(axis=0, keepdims=True)

    nblocks = N // _TILE
    idx3, loss_sum = pl.pallas_call(
        _vq_tc_kernel,
        grid=(nblocks,),
        in_specs=[
            pl.BlockSpec((_TILE, C), lambda i: (i, 0)),
            pl.BlockSpec((_TILE, 1), lambda i: (i, 0)),
            pl.BlockSpec((1, K), lambda i: (0, 0)),
            pl.BlockSpec((C, K), lambda i: (0, 0)),
        ],
        out_specs=[
            pl.BlockSpec((1, 1, _TILE), lambda i: (i, 0, 0)),
            pl.BlockSpec((1, 1, 1), lambda i: (i, 0, 0),
                         memory_space=pltpu.SMEM),
        ],
        out_shape=[
            jax.ShapeDtypeStruct((nblocks, 1, _TILE), jnp.int32),
            jax.ShapeDtypeStruct((nblocks, 1, 1), jnp.float32),
        ],
        compiler_params=pltpu.CompilerParams(
            dimension_semantics=("parallel",)),
    )(flat_x, x2, e2, embedding_table)

    et = jnp.pad(embedding_table.T, ((0, 0), (0, 128 - C)))
    q_flat = _sc_gather(et, idx3.reshape(1, N), N, 128)[:, :C]

    quantized = jnp.transpose(q_flat.reshape(B, H, W, C), (0, 3, 1, 2))
    loss = loss_sum.sum() / jnp.float32(N * C)
    indices = idx3.reshape(B, H * W)
    return (quantized, loss, loss, indices)
